# baseline (speedup 1.0000x reference)
import functools

import jax
import jax.numpy as jnp
from jax.experimental import pallas as pl
from jax.experimental.pallas import tpu as pltpu

_N_FRAGS = 16384
_IN_DIM = 4096
_N_EXPERTS = 64
_TOP_K = 8
_BLOCK_ROWS = 1024
_GRID = _N_FRAGS // _BLOCK_ROWS


def _router_block(x_ref, wt_ref, w_out_ref, part_ref):
    logits = jnp.dot(
        x_ref[...].astype(jnp.bfloat16),
        wt_ref[...].astype(jnp.bfloat16),
        preferred_element_type=jnp.float32,
    )

    col = jax.lax.broadcasted_iota(jnp.int32, logits.shape, 1)
    work = logits
    sel = jnp.zeros(logits.shape, dtype=jnp.bool_)
    row_max = jnp.max(logits, axis=-1, keepdims=True)
    for _ in range(_TOP_K):
        m = jnp.max(work, axis=-1, keepdims=True)
        eq = work == m
        first = jnp.min(jnp.where(eq, col, _N_EXPERTS), axis=-1, keepdims=True)
        hit = col == first
        sel = jnp.logical_or(sel, hit)
        work = jnp.where(hit, -jnp.inf, work)

    e = jnp.where(sel, jnp.exp(logits - row_max), 0.0)
    weights = e / jnp.sum(e, axis=-1, keepdims=True)
    w_out_ref[...] = weights
    part_ref[...] = jnp.sum(weights, axis=0, keepdims=True)[None] * (1.0 / _N_FRAGS)


@functools.partial(jax.jit)
def kernel(x, W):
    wt = W.T
    weights, parts = pl.pallas_call(
        _router_block,
        grid=(_GRID,),
        in_specs=[
            pl.BlockSpec((_BLOCK_ROWS, _IN_DIM), lambda i: (i, 0)),
            pl.BlockSpec((_IN_DIM, _N_EXPERTS), lambda i: (0, 0)),
        ],
        out_specs=[
            pl.BlockSpec((_BLOCK_ROWS, _N_EXPERTS), lambda i: (i, 0)),
            pl.BlockSpec((1, 1, _N_EXPERTS), lambda i: (i, 0, 0)),
        ],
        out_shape=[
            jax.ShapeDtypeStruct((_N_FRAGS, _N_EXPERTS), jnp.float32),
            jax.ShapeDtypeStruct((_GRID, 1, _N_EXPERTS), jnp.float32),
        ],
        compiler_params=pltpu.CompilerParams(
            dimension_semantics=("parallel",),
        ),
    )(x, wt)
    return weights, parts.sum(axis=(0, 1))


# speedup vs baseline: 1.1458x; 1.1458x over previous
import functools

import jax
import jax.numpy as jnp
from jax.experimental import pallas as pl
from jax.experimental.pallas import tpu as pltpu

_N_FRAGS = 16384
_IN_DIM = 4096
_N_EXPERTS = 64
_TOP_K = 8
_BLOCK_ROWS = 1024
_GRID = _N_FRAGS // _BLOCK_ROWS


def _router_block(x_ref, wt_ref, w_out_ref, part_ref):
    logits = jnp.dot(
        x_ref[...].astype(jnp.bfloat16),
        wt_ref[...].astype(jnp.bfloat16),
        preferred_element_type=jnp.float32,
    )

    work = logits
    sel = jnp.zeros(logits.shape, dtype=jnp.bool_)
    row_max = None
    for t in range(_TOP_K):
        m = jnp.max(work, axis=-1, keepdims=True)
        if t == 0:
            row_max = m
        hit = work == m
        sel = jnp.logical_or(sel, hit)
        work = jnp.where(hit, -jnp.inf, work)

    e = jnp.where(sel, jnp.exp(logits - row_max), 0.0)
    weights = e / jnp.sum(e, axis=-1, keepdims=True)
    w_out_ref[...] = weights
    part_ref[...] = jnp.sum(weights, axis=0, keepdims=True)[None] * (1.0 / _N_FRAGS)


@functools.partial(jax.jit)
def kernel(x, W):
    wt = W.T
    weights, parts = pl.pallas_call(
        _router_block,
        grid=(_GRID,),
        in_specs=[
            pl.BlockSpec((_BLOCK_ROWS, _IN_DIM), lambda i: (i, 0)),
            pl.BlockSpec((_IN_DIM, _N_EXPERTS), lambda i: (0, 0)),
        ],
        out_specs=[
            pl.BlockSpec((_BLOCK_ROWS, _N_EXPERTS), lambda i: (i, 0)),
            pl.BlockSpec((1, 1, _N_EXPERTS), lambda i: (i, 0, 0)),
        ],
        out_shape=[
            jax.ShapeDtypeStruct((_N_FRAGS, _N_EXPERTS), jnp.float32),
            jax.ShapeDtypeStruct((_GRID, 1, _N_EXPERTS), jnp.float32),
        ],
        compiler_params=pltpu.CompilerParams(
            dimension_semantics=("parallel",),
        ),
    )(x, wt)
    return weights, parts.sum(axis=(0, 1))
